# Initial kernel scaffold; baseline (speedup 1.0000x reference)
import functools

import jax
import jax.numpy as jnp
from jax import lax
from jax.experimental import pallas as pl
from jax.experimental.pallas import tpu as pltpu
from jax.experimental.pallas import tpu_sc as plsc

N_USERS = 5000
N_ITEMS = 5000
E = 320000
D = 128
B = 4096
NUM_LAYERS = 3

NC = 2
NS = 16
L = 16

NPAD = 5120
ROWS_PER_SUB = NPAD // NS
C = 128
EP = 20480
E_PAD = EP * NS
NCHUNK = EP // C
UNROLL = 4

B_PER_W = B // (NC * NS)


def _scale_rows(rows_ref, norm_ref, n_edges):
    def body(g, carry):
        for u in range(UNROLL):
            e = g * UNROLL + u
            evec = jnp.full((L,), 0, jnp.int32) + e
            nb = plsc.load_gather(norm_ref, [evec])
            for d in range(D // L):
                sl = pl.ds(d * L, L)
                rows_ref[e, sl] = rows_ref[e, sl] * nb
        return carry
    lax.fori_loop(0, n_edges // UNROLL, body, 0)


def _layer_body(hu, hi, nui, niu, src, dst, new_u, new_i,
                acc_sh, idxg_v, idxs_v, norm_v, rows_v, copy_buf, sem):
    c = lax.axis_index("c")
    s = lax.axis_index("s")

    z16 = jnp.zeros((L,), jnp.float32)
    def zbody(r, carry):
        for d in range(D // L):
            copy_buf[r, pl.ds(d * L, L)] = z16
        return carry
    lax.fori_loop(0, ROWS_PER_SUB, zbody, 0)
    pltpu.sync_copy(copy_buf, acc_sh.at[pl.ds(s * ROWS_PER_SUB, ROWS_PER_SUB)])
    plsc.subcore_barrier()

    def do_dir(table, gidx, sidx, norm, out):
        def chunk_body(k, carry):
            base = s * EP + k * C
            pltpu.sync_copy(gidx.at[pl.ds(base, C)], idxg_v)
            pltpu.sync_copy(sidx.at[pl.ds(base, C)], idxs_v)
            pltpu.sync_copy(norm.at[pl.ds(base, C)], norm_v)
            pltpu.async_copy(table.at[idxg_v], rows_v, sem).wait()
            _scale_rows(rows_v, norm_v, C)
            pltpu.sync_copy(rows_v, acc_sh.at[idxs_v], add=True)
            return carry
        lax.fori_loop(0, NCHUNK, chunk_body, 0)
        plsc.subcore_barrier()
        sl = pl.ds(s * ROWS_PER_SUB, ROWS_PER_SUB)
        pltpu.sync_copy(acc_sh.at[sl], copy_buf)
        pltpu.sync_copy(copy_buf, out.at[sl])

    @pl.when(c == 0)
    def _():
        do_dir(hu, src, dst, nui, new_i)

    @pl.when(c == 1)
    def _():
        do_dir(hi, dst, src, niu, new_u)


_layer_call = functools.partial(
    pl.kernel,
    out_type=(
        jax.ShapeDtypeStruct((NPAD, D), jnp.float32),
        jax.ShapeDtypeStruct((NPAD, D), jnp.float32),
    ),
    mesh=plsc.VectorSubcoreMesh(core_axis_name="c", subcore_axis_name="s"),
    scratch_types=[
        pltpu.VMEM_SHARED((NPAD, D), jnp.float32),
        pltpu.VMEM((C,), jnp.int32),
        pltpu.VMEM((C,), jnp.int32),
        pltpu.VMEM((C,), jnp.float32),
        pltpu.VMEM((C, D), jnp.float32),
        pltpu.VMEM((ROWS_PER_SUB, D), jnp.float32),
        pltpu.SemaphoreType.DMA,
    ],
)


def _layer(hu, hi, nui, niu, src, dst):
    return _layer_call(_layer_body)(hu, hi, nui, niu, src, dst)


def _readout_body(hu0, hu1, hu2, hu3, hi0, hi1, hi2, hi3,
                  users, pos, neg, u_out, p_out, n_out,
                  idx_v, rows_a, rows_b, sem):
    c = lax.axis_index("c")
    s = lax.axis_index("s")
    wid = s * NC + c

    def do_read(t0, t1, t2, t3, idx_hbm, out_hbm):
        base = wid * B_PER_W
        pltpu.sync_copy(idx_hbm.at[pl.ds(base, B_PER_W)], idx_v)
        pltpu.async_copy(t0.at[idx_v], rows_a, sem).wait()
        for t in (t1, t2, t3):
            pltpu.async_copy(t.at[idx_v], rows_b, sem).wait()
            def addbody(r, carry):
                for d in range(D // L):
                    sl = pl.ds(d * L, L)
                    rows_a[r, sl] = rows_a[r, sl] + rows_b[r, sl]
                return carry
            lax.fori_loop(0, B_PER_W, addbody, 0)
        def scbody(r, carry):
            for d in range(D // L):
                sl = pl.ds(d * L, L)
                rows_a[r, sl] = rows_a[r, sl] * 0.25
            return carry
        lax.fori_loop(0, B_PER_W, scbody, 0)
        pltpu.sync_copy(rows_a, out_hbm.at[pl.ds(base, B_PER_W)])

    do_read(hu0, hu1, hu2, hu3, users, u_out)
    do_read(hi0, hi1, hi2, hi3, pos, p_out)
    do_read(hi0, hi1, hi2, hi3, neg, n_out)


_readout_call = functools.partial(
    pl.kernel,
    out_type=(
        jax.ShapeDtypeStruct((B, D), jnp.float32),
        jax.ShapeDtypeStruct((B, D), jnp.float32),
        jax.ShapeDtypeStruct((B, D), jnp.float32),
    ),
    mesh=plsc.VectorSubcoreMesh(core_axis_name="c", subcore_axis_name="s"),
    scratch_types=[
        pltpu.VMEM((B_PER_W,), jnp.int32),
        pltpu.VMEM((B_PER_W, D), jnp.float32),
        pltpu.VMEM((B_PER_W, D), jnp.float32),
        pltpu.SemaphoreType.DMA,
    ],
)


def kernel(user_emb, item_emb, norm_ui, norm_iu, edge_src_user,
           edge_dst_item, users, pos_items, neg_items):
    pad_e = E_PAD - E
    src = jnp.concatenate([edge_src_user, jnp.zeros((pad_e,), jnp.int32)])
    dst = jnp.concatenate([edge_dst_item, jnp.zeros((pad_e,), jnp.int32)])
    nui = jnp.concatenate([norm_ui, jnp.zeros((pad_e,), jnp.float32)])
    niu = jnp.concatenate([norm_iu, jnp.zeros((pad_e,), jnp.float32)])

    zrows = jnp.zeros((NPAD - N_USERS, D), jnp.float32)
    hu0 = jnp.concatenate([user_emb, zrows], axis=0)
    hi0 = jnp.concatenate([item_emb, zrows], axis=0)

    hu1, hi1 = _layer(hu0, hi0, nui, niu, src, dst)
    hu2, hi2 = _layer(hu1, hi1, nui, niu, src, dst)
    hu3, hi3 = _layer(hu2, hi2, nui, niu, src, dst)

    u_g, p_g, n_g = _readout_call(_readout_body)(
        hu0, hu1, hu2, hu3, hi0, hi1, hi2, hi3,
        users, pos_items, neg_items)
    return (u_g, p_g, n_g)


# speedup vs baseline: 2.5464x; 2.5464x over previous
import functools

import jax
import jax.numpy as jnp
from jax import lax
from jax.experimental import pallas as pl
from jax.experimental.pallas import tpu as pltpu
from jax.experimental.pallas import tpu_sc as plsc

N_USERS = 5000
N_ITEMS = 5000
E = 320000
D = 128
B = 4096
NUM_LAYERS = 3

NC = 2
NS = 16
L = 16

NPAD = 5120
ROWS_PER_SUB = NPAD // NS
C = 128
EP = 20480
E_PAD = EP * NS
NCHUNK = EP // C
UNROLL = 4

B_PER_W = B // (NC * NS)


_BCAST_DNUMS = lax.GatherDimensionNumbers(
    offset_dims=(), collapsed_slice_dims=(0,), start_index_map=(0,))


def _bcast_lane(vec16, j):
    idx = jnp.full((L, 1), j, jnp.int32)
    return lax.gather(vec16, idx, _BCAST_DNUMS, (1,),
                      mode=lax.GatherScatterMode.PROMISE_IN_BOUNDS)


def _scale_rows(rows_ref, norm_ref, n_edges):
    def body(g, carry):
        norms16 = norm_ref[pl.ds(g * L, L)]
        for j in range(L):
            e = g * L + j
            nb = _bcast_lane(norms16, j)
            for d in range(D // L):
                sl = pl.ds(d * L, L)
                rows_ref[e, sl] = rows_ref[e, sl] * nb
        return carry
    lax.fori_loop(0, n_edges // L, body, 0)


def _layer_body(hu, hi, nui, niu, src, dst, new_u, new_i,
                acc_sh, idxg_v, idxs_v, norm_v, rows_v, copy_buf, sem):
    c = lax.axis_index("c")
    s = lax.axis_index("s")

    z16 = jnp.zeros((L,), jnp.float32)
    def zbody(r, carry):
        for d in range(D // L):
            copy_buf[r, pl.ds(d * L, L)] = z16
        return carry
    lax.fori_loop(0, ROWS_PER_SUB, zbody, 0)
    pltpu.sync_copy(copy_buf, acc_sh.at[pl.ds(s * ROWS_PER_SUB, ROWS_PER_SUB)])
    plsc.subcore_barrier()

    def do_dir(table, gidx, sidx, norm, out):
        def chunk_body(k, carry):
            base = s * EP + k * C
            pltpu.sync_copy(gidx.at[pl.ds(base, C)], idxg_v)
            pltpu.sync_copy(sidx.at[pl.ds(base, C)], idxs_v)
            pltpu.sync_copy(norm.at[pl.ds(base, C)], norm_v)
            pltpu.async_copy(table.at[idxg_v], rows_v, sem).wait()
            _scale_rows(rows_v, norm_v, C)
            pltpu.sync_copy(rows_v, acc_sh.at[idxs_v], add=True)
            return carry
        lax.fori_loop(0, NCHUNK, chunk_body, 0)
        plsc.subcore_barrier()
        sl = pl.ds(s * ROWS_PER_SUB, ROWS_PER_SUB)
        pltpu.sync_copy(acc_sh.at[sl], copy_buf)
        pltpu.sync_copy(copy_buf, out.at[sl])

    @pl.when(c == 0)
    def _():
        do_dir(hu, src, dst, nui, new_i)

    @pl.when(c == 1)
    def _():
        do_dir(hi, dst, src, niu, new_u)


_layer_call = functools.partial(
    pl.kernel,
    out_type=(
        jax.ShapeDtypeStruct((NPAD, D), jnp.float32),
        jax.ShapeDtypeStruct((NPAD, D), jnp.float32),
    ),
    mesh=plsc.VectorSubcoreMesh(core_axis_name="c", subcore_axis_name="s"),
    scratch_types=[
        pltpu.VMEM_SHARED((NPAD, D), jnp.float32),
        pltpu.VMEM((C,), jnp.int32),
        pltpu.VMEM((C,), jnp.int32),
        pltpu.VMEM((C,), jnp.float32),
        pltpu.VMEM((C, D), jnp.float32),
        pltpu.VMEM((ROWS_PER_SUB, D), jnp.float32),
        pltpu.SemaphoreType.DMA,
    ],
)


def _layer(hu, hi, nui, niu, src, dst):
    return _layer_call(_layer_body)(hu, hi, nui, niu, src, dst)


def _readout_body(hu0, hu1, hu2, hu3, hi0, hi1, hi2, hi3,
                  users, pos, neg, u_out, p_out, n_out,
                  idx_v, rows_a, rows_b, sem):
    c = lax.axis_index("c")
    s = lax.axis_index("s")
    wid = s * NC + c

    def do_read(t0, t1, t2, t3, idx_hbm, out_hbm):
        base = wid * B_PER_W
        pltpu.sync_copy(idx_hbm.at[pl.ds(base, B_PER_W)], idx_v)
        pltpu.async_copy(t0.at[idx_v], rows_a, sem).wait()
        for t in (t1, t2, t3):
            pltpu.async_copy(t.at[idx_v], rows_b, sem).wait()
            def addbody(r, carry):
                for d in range(D // L):
                    sl = pl.ds(d * L, L)
                    rows_a[r, sl] = rows_a[r, sl] + rows_b[r, sl]
                return carry
            lax.fori_loop(0, B_PER_W, addbody, 0)
        def scbody(r, carry):
            for d in range(D // L):
                sl = pl.ds(d * L, L)
                rows_a[r, sl] = rows_a[r, sl] * 0.25
            return carry
        lax.fori_loop(0, B_PER_W, scbody, 0)
        pltpu.sync_copy(rows_a, out_hbm.at[pl.ds(base, B_PER_W)])

    do_read(hu0, hu1, hu2, hu3, users, u_out)
    do_read(hi0, hi1, hi2, hi3, pos, p_out)
    do_read(hi0, hi1, hi2, hi3, neg, n_out)


_readout_call = functools.partial(
    pl.kernel,
    out_type=(
        jax.ShapeDtypeStruct((B, D), jnp.float32),
        jax.ShapeDtypeStruct((B, D), jnp.float32),
        jax.ShapeDtypeStruct((B, D), jnp.float32),
    ),
    mesh=plsc.VectorSubcoreMesh(core_axis_name="c", subcore_axis_name="s"),
    scratch_types=[
        pltpu.VMEM((B_PER_W,), jnp.int32),
        pltpu.VMEM((B_PER_W, D), jnp.float32),
        pltpu.VMEM((B_PER_W, D), jnp.float32),
        pltpu.SemaphoreType.DMA,
    ],
)


def kernel(user_emb, item_emb, norm_ui, norm_iu, edge_src_user,
           edge_dst_item, users, pos_items, neg_items):
    pad_e = E_PAD - E
    src = jnp.concatenate([edge_src_user, jnp.zeros((pad_e,), jnp.int32)])
    dst = jnp.concatenate([edge_dst_item, jnp.zeros((pad_e,), jnp.int32)])
    nui = jnp.concatenate([norm_ui, jnp.zeros((pad_e,), jnp.float32)])
    niu = jnp.concatenate([norm_iu, jnp.zeros((pad_e,), jnp.float32)])

    zrows = jnp.zeros((NPAD - N_USERS, D), jnp.float32)
    hu0 = jnp.concatenate([user_emb, zrows], axis=0)
    hi0 = jnp.concatenate([item_emb, zrows], axis=0)

    hu1, hi1 = _layer(hu0, hi0, nui, niu, src, dst)
    hu2, hi2 = _layer(hu1, hi1, nui, niu, src, dst)
    hu3, hi3 = _layer(hu2, hi2, nui, niu, src, dst)

    u_g, p_g, n_g = _readout_call(_readout_body)(
        hu0, hu1, hu2, hu3, hi0, hi1, hi2, hi3,
        users, pos_items, neg_items)
    return (u_g, p_g, n_g)


# speedup vs baseline: 4.3568x; 1.7109x over previous
import functools

import jax
import jax.numpy as jnp
from jax import lax
from jax.experimental import pallas as pl
from jax.experimental.pallas import tpu as pltpu
from jax.experimental.pallas import tpu_sc as plsc

N_USERS = 5000
N_ITEMS = 5000
E = 320000
D = 128
B = 4096
NUM_LAYERS = 3

NC = 2
NS = 16
L = 16

NPAD = 5120
ROWS_PER_SUB = NPAD // NS
C = 128
EP = 20480
E_PAD = EP * NS
NCHUNK = EP // C
NCHUNK_ALL = NCHUNK * NS

B_PER_W = B // (NC * NS)

_BCAST_DNUMS = lax.GatherDimensionNumbers(
    offset_dims=(), collapsed_slice_dims=(0,), start_index_map=(0,))


def _bcast_lane(vec16, j):
    idx = jnp.full((L, 1), j, jnp.int32)
    return lax.gather(vec16, idx, _BCAST_DNUMS, (1,),
                      mode=lax.GatherScatterMode.PROMISE_IN_BOUNDS)


def _scale_rows(rows_ref, norm_ref):
    def body(g, carry):
        norms16 = norm_ref[pl.ds(g * L, L)]
        for j in range(L):
            e = g * L + j
            nb = _bcast_lane(norms16, j)
            for d in range(D // L):
                sl = pl.ds(d * L, L)
                rows_ref[e, sl] = rows_ref[e, sl] * nb
        return carry
    lax.fori_loop(0, C // L, body, 0)


def _layer_body(hu, hi, pk_ui, pk_iu, nm_ui, nm_iu, new_u, new_i,
                acc_sh, pk0, pk1, pk2, pk3, nm0, nm1, nm2, nm3,
                rows0, rows1, copy_buf,
                is0, is1, is2, is3, gs0, gs1, ss0, ss1):
    c = lax.axis_index("c")
    s = lax.axis_index("s")
    pk = (pk0, pk1, pk2, pk3)
    nm = (nm0, nm1, nm2, nm3)
    rows = (rows0, rows1)
    isem = (is0, is1, is2, is3)
    gsem = (gs0, gs1)
    ssem = (ss0, ss1)

    z16 = jnp.zeros((L,), jnp.float32)
    def zbody(r, carry):
        for d in range(D // L):
            copy_buf[r, pl.ds(d * L, L)] = z16
        return carry
    lax.fori_loop(0, ROWS_PER_SUB, zbody, 0)
    pltpu.sync_copy(copy_buf, acc_sh.at[pl.ds(s * ROWS_PER_SUB, ROWS_PER_SUB)])
    plsc.subcore_barrier()

    def do_dir(table, packed, norms, out):
        cbase = s * NCHUNK

        def idx_start(kc, slot):
            pltpu.make_async_copy(packed.at[cbase + kc], pk[slot],
                                  isem[slot]).start()
            pltpu.make_async_copy(norms.at[cbase + kc], nm[slot],
                                  isem[slot]).start()

        def idx_wait(kc, slot):
            pltpu.make_async_copy(packed.at[cbase + kc], pk[slot],
                                  isem[slot]).wait()
            pltpu.make_async_copy(norms.at[cbase + kc], nm[slot],
                                  isem[slot]).wait()

        def gat_start(pslot, rslot):
            pltpu.make_async_copy(table.at[pk[pslot].at[0]], rows[rslot],
                                  gsem[rslot]).start()

        def gat_wait(rslot):
            pltpu.make_async_copy(table.at[pk[0].at[0]], rows[rslot],
                                  gsem[rslot]).wait()

        def scat_start(pslot, rslot):
            pltpu.make_async_copy(rows[rslot], acc_sh.at[pk[pslot].at[1]],
                                  ssem[rslot]).start(add=True)

        def scat_wait(rslot):
            pltpu.make_async_copy(rows[rslot], acc_sh.at[pk[0].at[1]],
                                  ssem[rslot]).wait()

        idx_start(0, 0)
        idx_start(1, 1)
        idx_wait(0, 0)
        gat_start(0, 0)

        def body4(k4, carry):
            for b in range(4):
                k = k4 * 4 + b
                rb = b % 2
                rn = (b + 1) % 2
                pn1 = (b + 1) % 4
                pn2 = (b + 2) % 4

                @pl.when(k < NCHUNK - 1)
                def _():
                    idx_wait(k + 1, pn1)

                if b == 0:
                    @pl.when(k > 0)
                    def _():
                        scat_wait(rn)
                else:
                    scat_wait(rn)

                @pl.when(k < NCHUNK - 1)
                def _():
                    gat_start(pn1, rn)

                @pl.when(k < NCHUNK - 2)
                def _():
                    idx_start(k + 2, pn2)

                gat_wait(rb)
                _scale_rows(rows[rb], nm[b])
                scat_start(b, rb)
            return carry

        lax.fori_loop(0, NCHUNK // 4, body4, 0)
        scat_wait(1)
        plsc.subcore_barrier()
        sl = pl.ds(s * ROWS_PER_SUB, ROWS_PER_SUB)
        pltpu.sync_copy(acc_sh.at[sl], copy_buf)
        pltpu.sync_copy(copy_buf, out.at[sl])

    @pl.when(c == 0)
    def _():
        do_dir(hu, pk_ui, nm_ui, new_i)

    @pl.when(c == 1)
    def _():
        do_dir(hi, pk_iu, nm_iu, new_u)


_layer_call = functools.partial(
    pl.kernel,
    out_type=(
        jax.ShapeDtypeStruct((NPAD, D), jnp.float32),
        jax.ShapeDtypeStruct((NPAD, D), jnp.float32),
    ),
    mesh=plsc.VectorSubcoreMesh(core_axis_name="c", subcore_axis_name="s"),
    scratch_types=[
        pltpu.VMEM_SHARED((NPAD, D), jnp.float32),
        pltpu.VMEM((2, C), jnp.int32),
        pltpu.VMEM((2, C), jnp.int32),
        pltpu.VMEM((2, C), jnp.int32),
        pltpu.VMEM((2, C), jnp.int32),
        pltpu.VMEM((C,), jnp.float32),
        pltpu.VMEM((C,), jnp.float32),
        pltpu.VMEM((C,), jnp.float32),
        pltpu.VMEM((C,), jnp.float32),
        pltpu.VMEM((C, D), jnp.float32),
        pltpu.VMEM((C, D), jnp.float32),
        pltpu.VMEM((ROWS_PER_SUB, D), jnp.float32),
        pltpu.SemaphoreType.DMA,
        pltpu.SemaphoreType.DMA,
        pltpu.SemaphoreType.DMA,
        pltpu.SemaphoreType.DMA,
        pltpu.SemaphoreType.DMA,
        pltpu.SemaphoreType.DMA,
        pltpu.SemaphoreType.DMA,
        pltpu.SemaphoreType.DMA,
    ],
)


def _readout_body(hu0, hu1, hu2, hu3, hi0, hi1, hi2, hi3,
                  users, pos, neg, u_out, p_out, n_out,
                  idx_v, ra, rb, rc, sem0, sem1, sem2):
    c = lax.axis_index("c")
    s = lax.axis_index("s")
    wid = s * NC + c

    def add_into(dst, src):
        def body(r, carry):
            for d in range(D // L):
                sl = pl.ds(d * L, L)
                dst[r, sl] = dst[r, sl] + src[r, sl]
            return carry
        lax.fori_loop(0, B_PER_W, body, 0)

    def add_scale_into(dst, src):
        def body(r, carry):
            for d in range(D // L):
                sl = pl.ds(d * L, L)
                dst[r, sl] = (dst[r, sl] + src[r, sl]) * 0.25
            return carry
        lax.fori_loop(0, B_PER_W, body, 0)

    def do_read(t0, t1, t2, t3, idx_hbm, out_hbm):
        base = wid * B_PER_W
        pltpu.sync_copy(idx_hbm.at[pl.ds(base, B_PER_W)], idx_v)
        d0 = pltpu.async_copy(t0.at[idx_v], ra, sem0)
        d1 = pltpu.async_copy(t1.at[idx_v], rb, sem1)
        d2 = pltpu.async_copy(t2.at[idx_v], rc, sem2)
        d0.wait()
        d1.wait()
        add_into(ra, rb)
        d3 = pltpu.async_copy(t3.at[idx_v], rb, sem1)
        d2.wait()
        add_into(ra, rc)
        d3.wait()
        add_scale_into(ra, rb)
        pltpu.sync_copy(ra, out_hbm.at[pl.ds(base, B_PER_W)])

    do_read(hu0, hu1, hu2, hu3, users, u_out)
    do_read(hi0, hi1, hi2, hi3, pos, p_out)
    do_read(hi0, hi1, hi2, hi3, neg, n_out)


_readout_call = functools.partial(
    pl.kernel,
    out_type=(
        jax.ShapeDtypeStruct((B, D), jnp.float32),
        jax.ShapeDtypeStruct((B, D), jnp.float32),
        jax.ShapeDtypeStruct((B, D), jnp.float32),
    ),
    mesh=plsc.VectorSubcoreMesh(core_axis_name="c", subcore_axis_name="s"),
    scratch_types=[
        pltpu.VMEM((B_PER_W,), jnp.int32),
        pltpu.VMEM((B_PER_W, D), jnp.float32),
        pltpu.VMEM((B_PER_W, D), jnp.float32),
        pltpu.VMEM((B_PER_W, D), jnp.float32),
        pltpu.SemaphoreType.DMA,
        pltpu.SemaphoreType.DMA,
        pltpu.SemaphoreType.DMA,
    ],
)


def _pack_idx(gidx, sidx):
    g = gidx.reshape(NCHUNK_ALL, C)
    sct = sidx.reshape(NCHUNK_ALL, C)
    return jnp.stack([g, sct], axis=1)


def kernel(user_emb, item_emb, norm_ui, norm_iu, edge_src_user,
           edge_dst_item, users, pos_items, neg_items):
    pad_e = E_PAD - E
    src = jnp.concatenate([edge_src_user, jnp.zeros((pad_e,), jnp.int32)])
    dst = jnp.concatenate([edge_dst_item, jnp.zeros((pad_e,), jnp.int32)])
    nui = jnp.concatenate([norm_ui, jnp.zeros((pad_e,), jnp.float32)])
    niu = jnp.concatenate([norm_iu, jnp.zeros((pad_e,), jnp.float32)])

    pk_ui = _pack_idx(src, dst)
    pk_iu = _pack_idx(dst, src)
    nm_ui = nui.reshape(NCHUNK_ALL, C)
    nm_iu = niu.reshape(NCHUNK_ALL, C)

    zrows = jnp.zeros((NPAD - N_USERS, D), jnp.float32)
    hu0 = jnp.concatenate([user_emb, zrows], axis=0)
    hi0 = jnp.concatenate([item_emb, zrows], axis=0)

    layer = _layer_call(_layer_body)
    hu1, hi1 = layer(hu0, hi0, pk_ui, pk_iu, nm_ui, nm_iu)
    hu2, hi2 = layer(hu1, hi1, pk_ui, pk_iu, nm_ui, nm_iu)
    hu3, hi3 = layer(hu2, hi2, pk_ui, pk_iu, nm_ui, nm_iu)

    u_g, p_g, n_g = _readout_call(_readout_body)(
        hu0, hu1, hu2, hu3, hi0, hi1, hi2, hi3,
        users, pos_items, neg_items)
    return (u_g, p_g, n_g)


# speedup vs baseline: 4.3738x; 1.0039x over previous
import functools

import jax
import jax.numpy as jnp
from jax import lax
from jax.experimental import pallas as pl
from jax.experimental.pallas import tpu as pltpu
from jax.experimental.pallas import tpu_sc as plsc

N_USERS = 5000
N_ITEMS = 5000
E = 320000
D = 128
B = 4096
NUM_LAYERS = 3

NC = 2
NS = 16
L = 16

NPAD = 5120
ROWS_PER_SUB = NPAD // NS
C = 128
EP = 20480
E_PAD = EP * NS
NCHUNK = EP // C
NCHUNK_ALL = NCHUNK * NS

B_PER_W = B // (NC * NS)

_BCAST_DNUMS = lax.GatherDimensionNumbers(
    offset_dims=(), collapsed_slice_dims=(0,), start_index_map=(0,))


def _bcast_lane(vec16, j):
    idx = jnp.full((L, 1), j, jnp.int32)
    return lax.gather(vec16, idx, _BCAST_DNUMS, (1,),
                      mode=lax.GatherScatterMode.PROMISE_IN_BOUNDS)


def _scale_rows(rows_ref, norm_ref):
    def body(g, carry):
        norms16 = norm_ref[pl.ds(g * L, L)]
        for j in range(L):
            e = g * L + j
            nb = _bcast_lane(norms16, j)
            for d in range(D // L):
                sl = pl.ds(d * L, L)
                rows_ref[e, sl] = rows_ref[e, sl] * nb
        return carry
    lax.fori_loop(0, C // L, body, 0)


NPK = 8
NRW = 4
CB = 64


def _layer_body(hu, hi, pk_ui, pk_iu, nm_ui, nm_iu, new_u, new_i,
                acc_sh, pks, nms, rowss, copy_buf, isems, gsems, ssems):
    c = lax.axis_index("c")
    s = lax.axis_index("s")
    pk = tuple(pks)
    nm = tuple(nms)
    rows = tuple(rowss)
    isem = tuple(isems)
    gsem = tuple(gsems)
    ssem = tuple(ssems)

    z16 = jnp.zeros((L,), jnp.float32)
    def zbody(r, carry):
        for d in range(D // L):
            copy_buf[r, pl.ds(d * L, L)] = z16
        return carry
    lax.fori_loop(0, CB, zbody, 0)
    def zcp(t, carry):
        pltpu.sync_copy(copy_buf,
                        acc_sh.at[pl.ds(s * ROWS_PER_SUB + t * CB, CB)])
        return carry
    lax.fori_loop(0, ROWS_PER_SUB // CB, zcp, 0)
    plsc.subcore_barrier()

    def do_dir(table, packed, norms, out):
        cbase = s * NCHUNK

        def idx_start(kc, slot):
            pltpu.make_async_copy(packed.at[cbase + kc], pk[slot],
                                  isem[slot]).start()
            pltpu.make_async_copy(norms.at[cbase + kc], nm[slot],
                                  isem[slot]).start()

        def idx_wait(kc, slot):
            pltpu.make_async_copy(packed.at[cbase + kc], pk[slot],
                                  isem[slot]).wait()
            pltpu.make_async_copy(norms.at[cbase + kc], nm[slot],
                                  isem[slot]).wait()

        def gat_start(pslot, rslot):
            pltpu.make_async_copy(table.at[pk[pslot].at[0]], rows[rslot],
                                  gsem[rslot]).start()

        def gat_wait(rslot):
            pltpu.make_async_copy(table.at[pk[0].at[0]], rows[rslot],
                                  gsem[rslot]).wait()

        def scat_start(pslot, rslot):
            pltpu.make_async_copy(rows[rslot], acc_sh.at[pk[pslot].at[1]],
                                  ssem[rslot]).start(add=True)

        def scat_wait(rslot):
            pltpu.make_async_copy(rows[rslot], acc_sh.at[pk[0].at[1]],
                                  ssem[rslot]).wait()

        for j in range(4):
            idx_start(j, j)
        idx_wait(0, 0)
        gat_start(0, 0)
        idx_wait(1, 1)
        gat_start(1, 1)

        def body8(k8, carry):
            for b in range(NPK):
                k = k8 * NPK + b
                rb = b % NRW
                r2 = (b + 2) % NRW
                p2 = (b + 2) % NPK
                p4 = (b + 4) % NPK

                @pl.when(k < NCHUNK - 2)
                def _():
                    idx_wait(k + 2, p2)

                @pl.when(k >= 2)
                def _():
                    scat_wait(r2)

                @pl.when(k < NCHUNK - 2)
                def _():
                    gat_start(p2, r2)

                @pl.when(k < NCHUNK - 4)
                def _():
                    idx_start(k + 4, p4)

                gat_wait(rb)
                _scale_rows(rows[rb], nm[b])
                scat_start(b, rb)
            return carry

        lax.fori_loop(0, NCHUNK // NPK, body8, 0)
        scat_wait((NCHUNK - 2) % NRW)
        scat_wait((NCHUNK - 1) % NRW)
        plsc.subcore_barrier()
        def ocp(t, carry):
            sl = pl.ds(s * ROWS_PER_SUB + t * CB, CB)
            pltpu.sync_copy(acc_sh.at[sl], copy_buf)
            pltpu.sync_copy(copy_buf, out.at[sl])
            return carry
        lax.fori_loop(0, ROWS_PER_SUB // CB, ocp, 0)

    @pl.when(c == 0)
    def _():
        do_dir(hu, pk_ui, nm_ui, new_i)

    @pl.when(c == 1)
    def _():
        do_dir(hi, pk_iu, nm_iu, new_u)


_layer_call = functools.partial(
    pl.kernel,
    out_type=(
        jax.ShapeDtypeStruct((NPAD, D), jnp.float32),
        jax.ShapeDtypeStruct((NPAD, D), jnp.float32),
    ),
    mesh=plsc.VectorSubcoreMesh(core_axis_name="c", subcore_axis_name="s"),
    scratch_types=[
        pltpu.VMEM_SHARED((NPAD, D), jnp.float32),
        [pltpu.VMEM((2, C), jnp.int32) for _ in range(NPK)],
        [pltpu.VMEM((C,), jnp.float32) for _ in range(NPK)],
        [pltpu.VMEM((C, D), jnp.float32) for _ in range(NRW)],
        pltpu.VMEM((CB, D), jnp.float32),
        [pltpu.SemaphoreType.DMA for _ in range(NPK)],
        [pltpu.SemaphoreType.DMA for _ in range(NRW)],
        [pltpu.SemaphoreType.DMA for _ in range(NRW)],
    ],
)


def _readout_body(hu0, hu1, hu2, hu3, hi0, hi1, hi2, hi3,
                  users, pos, neg, u_out, p_out, n_out,
                  idx_v, ra, rb, rc, sem0, sem1, sem2):
    c = lax.axis_index("c")
    s = lax.axis_index("s")
    wid = s * NC + c

    def add_into(dst, src):
        def body(r, carry):
            for d in range(D // L):
                sl = pl.ds(d * L, L)
                dst[r, sl] = dst[r, sl] + src[r, sl]
            return carry
        lax.fori_loop(0, B_PER_W, body, 0)

    def add_scale_into(dst, src):
        def body(r, carry):
            for d in range(D // L):
                sl = pl.ds(d * L, L)
                dst[r, sl] = (dst[r, sl] + src[r, sl]) * 0.25
            return carry
        lax.fori_loop(0, B_PER_W, body, 0)

    def do_read(t0, t1, t2, t3, idx_hbm, out_hbm):
        base = wid * B_PER_W
        pltpu.sync_copy(idx_hbm.at[pl.ds(base, B_PER_W)], idx_v)
        d0 = pltpu.async_copy(t0.at[idx_v], ra, sem0)
        d1 = pltpu.async_copy(t1.at[idx_v], rb, sem1)
        d2 = pltpu.async_copy(t2.at[idx_v], rc, sem2)
        d0.wait()
        d1.wait()
        add_into(ra, rb)
        d3 = pltpu.async_copy(t3.at[idx_v], rb, sem1)
        d2.wait()
        add_into(ra, rc)
        d3.wait()
        add_scale_into(ra, rb)
        pltpu.sync_copy(ra, out_hbm.at[pl.ds(base, B_PER_W)])

    do_read(hu0, hu1, hu2, hu3, users, u_out)
    do_read(hi0, hi1, hi2, hi3, pos, p_out)
    do_read(hi0, hi1, hi2, hi3, neg, n_out)


_readout_call = functools.partial(
    pl.kernel,
    out_type=(
        jax.ShapeDtypeStruct((B, D), jnp.float32),
        jax.ShapeDtypeStruct((B, D), jnp.float32),
        jax.ShapeDtypeStruct((B, D), jnp.float32),
    ),
    mesh=plsc.VectorSubcoreMesh(core_axis_name="c", subcore_axis_name="s"),
    scratch_types=[
        pltpu.VMEM((B_PER_W,), jnp.int32),
        pltpu.VMEM((B_PER_W, D), jnp.float32),
        pltpu.VMEM((B_PER_W, D), jnp.float32),
        pltpu.VMEM((B_PER_W, D), jnp.float32),
        pltpu.SemaphoreType.DMA,
        pltpu.SemaphoreType.DMA,
        pltpu.SemaphoreType.DMA,
    ],
)


def _pack_idx(gidx, sidx):
    g = gidx.reshape(NCHUNK_ALL, C)
    sct = sidx.reshape(NCHUNK_ALL, C)
    return jnp.stack([g, sct], axis=1)


def kernel(user_emb, item_emb, norm_ui, norm_iu, edge_src_user,
           edge_dst_item, users, pos_items, neg_items):
    pad_e = E_PAD - E
    src = jnp.concatenate([edge_src_user, jnp.zeros((pad_e,), jnp.int32)])
    dst = jnp.concatenate([edge_dst_item, jnp.zeros((pad_e,), jnp.int32)])
    nui = jnp.concatenate([norm_ui, jnp.zeros((pad_e,), jnp.float32)])
    niu = jnp.concatenate([norm_iu, jnp.zeros((pad_e,), jnp.float32)])

    pk_ui = _pack_idx(src, dst)
    pk_iu = _pack_idx(dst, src)
    nm_ui = nui.reshape(NCHUNK_ALL, C)
    nm_iu = niu.reshape(NCHUNK_ALL, C)

    zrows = jnp.zeros((NPAD - N_USERS, D), jnp.float32)
    hu0 = jnp.concatenate([user_emb, zrows], axis=0)
    hi0 = jnp.concatenate([item_emb, zrows], axis=0)

    layer = _layer_call(_layer_body)
    hu1, hi1 = layer(hu0, hi0, pk_ui, pk_iu, nm_ui, nm_iu)
    hu2, hi2 = layer(hu1, hi1, pk_ui, pk_iu, nm_ui, nm_iu)
    hu3, hi3 = layer(hu2, hi2, pk_ui, pk_iu, nm_ui, nm_iu)

    u_g, p_g, n_g = _readout_call(_readout_body)(
        hu0, hu1, hu2, hu3, hi0, hi1, hi2, hi3,
        users, pos_items, neg_items)
    return (u_g, p_g, n_g)


# speedup vs baseline: 9.5639x; 2.1866x over previous
import functools

import jax
import jax.numpy as jnp
from jax import lax
from jax.experimental import pallas as pl
from jax.experimental.pallas import tpu as pltpu
from jax.experimental.pallas import tpu_sc as plsc

N_USERS = 5000
N_ITEMS = 5000
E = 320000
D = 128
B = 4096
NUM_LAYERS = 3

NC = 2
NS = 16
L = 16

NPAD = 5120
ROWS_PER_SUB = NPAD // NS
C = 128
EP = 20480
E_PAD = EP * NS
NCHUNK = EP // C
NCHUNK_ALL = NCHUNK * NS

B_PER_W = B // (NC * NS)

_BCAST_DNUMS = lax.GatherDimensionNumbers(
    offset_dims=(), collapsed_slice_dims=(0,), start_index_map=(0,))


def _bcast_lane(vec16, j):
    idx = jnp.full((L, 1), j, jnp.int32)
    return lax.gather(vec16, idx, _BCAST_DNUMS, (1,),
                      mode=lax.GatherScatterMode.PROMISE_IN_BOUNDS)


def _scale_rows(rows_ref, norm_ref):
    def body(g, carry):
        norms16 = norm_ref[pl.ds(g * L, L)]
        for j in range(L):
            e = g * L + j
            nb = _bcast_lane(norms16, j)
            for d in range(D // L):
                sl = pl.ds(d * L, L)
                rows_ref[e, sl] = rows_ref[e, sl] * nb
        return carry
    lax.fori_loop(0, C // L, body, 0)


NPK = 4
NRW = 2
CB = 64


def _layer_body(hu, hi, pk_ui, pk_iu, nm_ui, nm_iu, new_u, new_i,
                acc_sh, table_sh, pks, nms, rowss, copy_buf,
                isems, gsems, ssems):
    c = lax.axis_index("c")
    s = lax.axis_index("s")
    pk = tuple(pks)
    nm = tuple(nms)
    rows = tuple(rowss)
    isem = tuple(isems)
    gsem = tuple(gsems)
    ssem = tuple(ssems)

    z16 = jnp.zeros((L,), jnp.float32)
    def zbody(r, carry):
        for d in range(D // L):
            copy_buf[r, pl.ds(d * L, L)] = z16
        return carry
    lax.fori_loop(0, CB, zbody, 0)
    def zcp(t, carry):
        pltpu.sync_copy(copy_buf,
                        acc_sh.at[pl.ds(s * ROWS_PER_SUB + t * CB, CB)])
        return carry
    lax.fori_loop(0, ROWS_PER_SUB // CB, zcp, 0)
    plsc.subcore_barrier()

    def do_dir(table, packed, norms, out):
        cbase = s * NCHUNK

        def tcp(t, carry):
            sl = pl.ds(s * ROWS_PER_SUB + t * CB, CB)
            pltpu.sync_copy(table.at[sl], copy_buf)
            pltpu.sync_copy(copy_buf, table_sh.at[sl])
            return carry
        lax.fori_loop(0, ROWS_PER_SUB // CB, tcp, 0)
        plsc.subcore_barrier()

        def idx_start(kc, slot):
            pltpu.make_async_copy(packed.at[cbase + kc], pk[slot],
                                  isem[slot]).start()
            pltpu.make_async_copy(norms.at[cbase + kc], nm[slot],
                                  isem[slot]).start()

        def idx_wait(kc, slot):
            pltpu.make_async_copy(packed.at[cbase + kc], pk[slot],
                                  isem[slot]).wait()
            pltpu.make_async_copy(norms.at[cbase + kc], nm[slot],
                                  isem[slot]).wait()

        def gat_start(pslot, rslot):
            pltpu.make_async_copy(table_sh.at[pk[pslot].at[0]], rows[rslot],
                                  gsem[rslot]).start()

        def gat_wait(rslot):
            pltpu.make_async_copy(table_sh.at[pk[0].at[0]], rows[rslot],
                                  gsem[rslot]).wait()

        def scat_start(pslot, rslot):
            pltpu.make_async_copy(rows[rslot], acc_sh.at[pk[pslot].at[1]],
                                  ssem[rslot]).start(add=True)

        def scat_wait(rslot):
            pltpu.make_async_copy(rows[rslot], acc_sh.at[pk[0].at[1]],
                                  ssem[rslot]).wait()

        idx_start(0, 0)
        idx_start(1, 1)
        idx_wait(0, 0)
        gat_start(0, 0)

        def body4(k4, carry):
            for b in range(NPK):
                k = k4 * NPK + b
                rb = b % NRW
                rn = (b + 1) % NRW
                pn1 = (b + 1) % NPK
                pn2 = (b + 2) % NPK

                @pl.when(k < NCHUNK - 1)
                def _():
                    idx_wait(k + 1, pn1)

                if b == 0:
                    @pl.when(k > 0)
                    def _():
                        scat_wait(rn)
                else:
                    scat_wait(rn)

                @pl.when(k < NCHUNK - 1)
                def _():
                    gat_start(pn1, rn)

                @pl.when(k < NCHUNK - 2)
                def _():
                    idx_start(k + 2, pn2)

                gat_wait(rb)
                _scale_rows(rows[rb], nm[b])
                scat_start(b, rb)
            return carry

        lax.fori_loop(0, NCHUNK // NPK, body4, 0)
        scat_wait((NCHUNK - 1) % NRW)
        plsc.subcore_barrier()
        def ocp(t, carry):
            sl = pl.ds(s * ROWS_PER_SUB + t * CB, CB)
            pltpu.sync_copy(acc_sh.at[sl], copy_buf)
            pltpu.sync_copy(copy_buf, out.at[sl])
            return carry
        lax.fori_loop(0, ROWS_PER_SUB // CB, ocp, 0)

    @pl.when(c == 0)
    def _():
        do_dir(hu, pk_ui, nm_ui, new_i)

    @pl.when(c == 1)
    def _():
        do_dir(hi, pk_iu, nm_iu, new_u)


_layer_call = functools.partial(
    pl.kernel,
    out_type=(
        jax.ShapeDtypeStruct((NPAD, D), jnp.float32),
        jax.ShapeDtypeStruct((NPAD, D), jnp.float32),
    ),
    mesh=plsc.VectorSubcoreMesh(core_axis_name="c", subcore_axis_name="s"),
    scratch_types=[
        pltpu.VMEM_SHARED((NPAD, D), jnp.float32),
        pltpu.VMEM_SHARED((NPAD, D), jnp.float32),
        [pltpu.VMEM((2, C), jnp.int32) for _ in range(NPK)],
        [pltpu.VMEM((C,), jnp.float32) for _ in range(NPK)],
        [pltpu.VMEM((C, D), jnp.float32) for _ in range(NRW)],
        pltpu.VMEM((CB, D), jnp.float32),
        [pltpu.SemaphoreType.DMA for _ in range(NPK)],
        [pltpu.SemaphoreType.DMA for _ in range(NRW)],
        [pltpu.SemaphoreType.DMA for _ in range(NRW)],
    ],
)


def _readout_body(hu0, hu1, hu2, hu3, hi0, hi1, hi2, hi3,
                  users, pos, neg, u_out, p_out, n_out,
                  idx_v, ra, rb, rc, sem0, sem1, sem2):
    c = lax.axis_index("c")
    s = lax.axis_index("s")
    wid = s * NC + c

    def add_into(dst, src):
        def body(r, carry):
            for d in range(D // L):
                sl = pl.ds(d * L, L)
                dst[r, sl] = dst[r, sl] + src[r, sl]
            return carry
        lax.fori_loop(0, B_PER_W, body, 0)

    def add_scale_into(dst, src):
        def body(r, carry):
            for d in range(D // L):
                sl = pl.ds(d * L, L)
                dst[r, sl] = (dst[r, sl] + src[r, sl]) * 0.25
            return carry
        lax.fori_loop(0, B_PER_W, body, 0)

    def do_read(t0, t1, t2, t3, idx_hbm, out_hbm):
        base = wid * B_PER_W
        pltpu.sync_copy(idx_hbm.at[pl.ds(base, B_PER_W)], idx_v)
        d0 = pltpu.async_copy(t0.at[idx_v], ra, sem0)
        d1 = pltpu.async_copy(t1.at[idx_v], rb, sem1)
        d2 = pltpu.async_copy(t2.at[idx_v], rc, sem2)
        d0.wait()
        d1.wait()
        add_into(ra, rb)
        d3 = pltpu.async_copy(t3.at[idx_v], rb, sem1)
        d2.wait()
        add_into(ra, rc)
        d3.wait()
        add_scale_into(ra, rb)
        pltpu.sync_copy(ra, out_hbm.at[pl.ds(base, B_PER_W)])

    do_read(hu0, hu1, hu2, hu3, users, u_out)
    do_read(hi0, hi1, hi2, hi3, pos, p_out)
    do_read(hi0, hi1, hi2, hi3, neg, n_out)


_readout_call = functools.partial(
    pl.kernel,
    out_type=(
        jax.ShapeDtypeStruct((B, D), jnp.float32),
        jax.ShapeDtypeStruct((B, D), jnp.float32),
        jax.ShapeDtypeStruct((B, D), jnp.float32),
    ),
    mesh=plsc.VectorSubcoreMesh(core_axis_name="c", subcore_axis_name="s"),
    scratch_types=[
        pltpu.VMEM((B_PER_W,), jnp.int32),
        pltpu.VMEM((B_PER_W, D), jnp.float32),
        pltpu.VMEM((B_PER_W, D), jnp.float32),
        pltpu.VMEM((B_PER_W, D), jnp.float32),
        pltpu.SemaphoreType.DMA,
        pltpu.SemaphoreType.DMA,
        pltpu.SemaphoreType.DMA,
    ],
)


def _pack_idx(gidx, sidx):
    g = gidx.reshape(NCHUNK_ALL, C)
    sct = sidx.reshape(NCHUNK_ALL, C)
    return jnp.stack([g, sct], axis=1)


def kernel(user_emb, item_emb, norm_ui, norm_iu, edge_src_user,
           edge_dst_item, users, pos_items, neg_items):
    pad_e = E_PAD - E
    src = jnp.concatenate([edge_src_user, jnp.zeros((pad_e,), jnp.int32)])
    dst = jnp.concatenate([edge_dst_item, jnp.zeros((pad_e,), jnp.int32)])
    nui = jnp.concatenate([norm_ui, jnp.zeros((pad_e,), jnp.float32)])
    niu = jnp.concatenate([norm_iu, jnp.zeros((pad_e,), jnp.float32)])

    pk_ui = _pack_idx(src, dst)
    pk_iu = _pack_idx(dst, src)
    nm_ui = nui.reshape(NCHUNK_ALL, C)
    nm_iu = niu.reshape(NCHUNK_ALL, C)

    zrows = jnp.zeros((NPAD - N_USERS, D), jnp.float32)
    hu0 = jnp.concatenate([user_emb, zrows], axis=0)
    hi0 = jnp.concatenate([item_emb, zrows], axis=0)

    layer = _layer_call(_layer_body)
    hu1, hi1 = layer(hu0, hi0, pk_ui, pk_iu, nm_ui, nm_iu)
    hu2, hi2 = layer(hu1, hi1, pk_ui, pk_iu, nm_ui, nm_iu)
    hu3, hi3 = layer(hu2, hi2, pk_ui, pk_iu, nm_ui, nm_iu)

    u_g, p_g, n_g = _readout_call(_readout_body)(
        hu0, hu1, hu2, hu3, hi0, hi1, hi2, hi3,
        users, pos_items, neg_items)
    return (u_g, p_g, n_g)
